# baseline (device time: 22038 ns/iter reference)
import os

import jax
import jax.numpy as jnp
from jax import lax
from jax.experimental import pallas as pl
from jax.experimental.pallas import tpu as pltpu

N_DEV = 8

ORDER = os.environ.get("A2A_ORDER", "xor")
CHUNKS = int(os.environ.get("A2A_CHUNKS", "1"))
NSENDS = int(os.environ.get("A2A_NSENDS", "7"))
HALF = int(os.environ.get("A2A_HALF", "1"))
BARRIER = os.environ.get("A2A_BARRIER", "auto")
INTERLEAVE = os.environ.get("A2A_INTERLEAVE", "0") == "1"

_DELTAS = [
    (1, 0, 0), (0, 1, 0), (0, 0, 1),
    (1, 1, 0), (0, 1, 1), (1, 0, 1),
    (1, 1, 1),
]
if os.environ.get("A2A_FAR_FIRST", "0") == "1":
    _DELTAS = _DELTAS[::-1]


def _xor_partner(me, delta):
    dx, dy, dz = delta
    z = me // 4
    r = me - 4 * z
    x = jnp.where((r == 1) | (r == 2), 1, 0)
    y = jnp.where(r >= 2, 1, 0)
    px = x + dx - 2 * x * dx
    py = y + dy - 2 * y * dy
    pz = z + dz - 2 * z * dz
    p4 = 3 * py + px - 2 * px * py
    return 4 * pz + p4


def kernel(x):
    m, n_total = x.shape
    blk = n_total // N_DEV
    out_rows = m * N_DEV
    mc = m // CHUNKS // HALF

    def body(x_ref, out_ref, send_sems, recv_sems, hsems):
        me = lax.axis_index("i")
        rdmas = []

        def start_send(j, slot):
            for c in range(CHUNKS):
                s = slot * CHUNKS + c
                rdma = pltpu.make_async_remote_copy(
                    src_ref=x_ref.at[pl.ds(c * mc, mc), pl.ds(j * blk, blk)],
                    dst_ref=out_ref.at[pl.ds(me * m + c * mc, mc), :],
                    send_sem=send_sems.at[s],
                    recv_sem=recv_sems.at[s],
                    device_id=(j,),
                    device_id_type=pl.DeviceIdType.MESH,
                )
                rdma.start()
                rdmas.append(rdma)

        if BARRIER == "explicit":
            barrier_sem = pltpu.get_barrier_semaphore()
            for k in range(1, N_DEV):
                nbr = lax.rem(me + k, N_DEV)
                pl.semaphore_signal(
                    barrier_sem, inc=1,
                    device_id=(nbr,), device_id_type=pl.DeviceIdType.MESH,
                )
            pl.semaphore_wait(barrier_sem, N_DEV - 1)
        elif BARRIER == "hyper":
            barrier_sem = pltpu.get_barrier_semaphore()
            phase_sems = [barrier_sem, hsems.at[0], hsems.at[1]]
            groups = [
                [(1, 0, 0)],
                [(1, 1, 0), (0, 1, 0)],
                [(1, 1, 1), (1, 0, 1), (0, 1, 1), (0, 0, 1)],
            ]
            for p, pdelta in enumerate([(1, 0, 0), (0, 1, 0), (0, 0, 1)]):
                partner = _xor_partner(me, pdelta)
                pl.semaphore_signal(
                    phase_sems[p], inc=1,
                    device_id=(partner,), device_id_type=pl.DeviceIdType.MESH,
                )
                pl.semaphore_wait(phase_sems[p], 1)
                if INTERLEAVE:
                    for d in groups[p]:
                        start_send(_xor_partner(me, d), _DELTAS.index(d))

        if not (BARRIER == "hyper" and INTERLEAVE):
            for k in range(1, 1 + NSENDS):
                if ORDER == "xor":
                    j = _xor_partner(me, _DELTAS[k - 1])
                else:
                    j = lax.rem(me + k, N_DEV)
                start_send(j, k - 1)

        out_ref[pl.ds(me * m, m), :] = x_ref[:, pl.ds(me * blk, blk)]

        for rdma in rdmas:
            rdma.wait()

    nsem = (N_DEV - 1) * CHUNKS
    return pl.pallas_call(
        body,
        out_shape=jax.ShapeDtypeStruct((out_rows, blk), x.dtype),
        in_specs=[pl.BlockSpec(memory_space=pltpu.VMEM)],
        out_specs=pl.BlockSpec(memory_space=pltpu.VMEM),
        scratch_shapes=[
            pltpu.SemaphoreType.DMA((nsem,)),
            pltpu.SemaphoreType.DMA((nsem,)),
            pltpu.SemaphoreType.REGULAR((2,)),
        ],
        compiler_params=pltpu.CompilerParams(collective_id=0),
    )(x)


# device time: 21917 ns/iter; 1.0055x vs baseline; 1.0055x over previous
import os

import jax
import jax.numpy as jnp
from jax import lax
from jax.experimental import pallas as pl
from jax.experimental.pallas import tpu as pltpu

N_DEV = 8

ORDER = os.environ.get("A2A_ORDER", "xor")
CHUNKS = int(os.environ.get("A2A_CHUNKS", "1"))
NSENDS = int(os.environ.get("A2A_NSENDS", "7"))
HALF = int(os.environ.get("A2A_HALF", "1"))
BARRIER = os.environ.get("A2A_BARRIER", "auto")
MEM = os.environ.get("A2A_MEM", "vmem")
INTERLEAVE = os.environ.get("A2A_INTERLEAVE", "0") == "1"

_DELTAS = [
    (1, 0, 0), (0, 1, 0), (0, 0, 1),
    (1, 1, 0), (0, 1, 1), (1, 0, 1),
    (1, 1, 1),
]
if os.environ.get("A2A_FAR_FIRST", "0") == "1":
    _DELTAS = _DELTAS[::-1]


def _xor_partner(me, delta):
    dx, dy, dz = delta
    z = me // 4
    r = me - 4 * z
    x = jnp.where((r == 1) | (r == 2), 1, 0)
    y = jnp.where(r >= 2, 1, 0)
    px = x + dx - 2 * x * dx
    py = y + dy - 2 * y * dy
    pz = z + dz - 2 * z * dz
    p4 = 3 * py + px - 2 * px * py
    return 4 * pz + p4


def kernel(x):
    m, n_total = x.shape
    blk = n_total // N_DEV
    out_rows = m * N_DEV
    mc = m // CHUNKS // HALF

    def body(x_ref, out_ref, send_sems, recv_sems, hsems, local_sem):
        me = lax.axis_index("i")
        rdmas = []

        def start_send(j, slot):
            for c in range(CHUNKS):
                s = slot * CHUNKS + c
                rdma = pltpu.make_async_remote_copy(
                    src_ref=x_ref.at[pl.ds(c * mc, mc), pl.ds(j * blk, blk)],
                    dst_ref=out_ref.at[pl.ds(me * m + c * mc, mc), :],
                    send_sem=send_sems.at[s],
                    recv_sem=recv_sems.at[s],
                    device_id=(j,),
                    device_id_type=pl.DeviceIdType.MESH,
                )
                rdma.start()
                rdmas.append(rdma)

        if BARRIER == "explicit":
            barrier_sem = pltpu.get_barrier_semaphore()
            for k in range(1, N_DEV):
                nbr = lax.rem(me + k, N_DEV)
                pl.semaphore_signal(
                    barrier_sem, inc=1,
                    device_id=(nbr,), device_id_type=pl.DeviceIdType.MESH,
                )
            pl.semaphore_wait(barrier_sem, N_DEV - 1)
        elif BARRIER == "hyper":
            barrier_sem = pltpu.get_barrier_semaphore()
            phase_sems = [barrier_sem, hsems.at[0], hsems.at[1]]
            groups = [
                [(1, 0, 0)],
                [(1, 1, 0), (0, 1, 0)],
                [(1, 1, 1), (1, 0, 1), (0, 1, 1), (0, 0, 1)],
            ]
            for p, pdelta in enumerate([(1, 0, 0), (0, 1, 0), (0, 0, 1)]):
                partner = _xor_partner(me, pdelta)
                pl.semaphore_signal(
                    phase_sems[p], inc=1,
                    device_id=(partner,), device_id_type=pl.DeviceIdType.MESH,
                )
                pl.semaphore_wait(phase_sems[p], 1)
                if INTERLEAVE:
                    for d in groups[p]:
                        start_send(_xor_partner(me, d), _DELTAS.index(d))

        if not (BARRIER == "hyper" and INTERLEAVE):
            for k in range(1, 1 + NSENDS):
                if ORDER == "xor":
                    j = _xor_partner(me, _DELTAS[k - 1])
                else:
                    j = lax.rem(me + k, N_DEV)
                start_send(j, k - 1)

        if MEM == "any":
            diag = pltpu.make_async_copy(
                x_ref.at[:, pl.ds(me * blk, blk)],
                out_ref.at[pl.ds(me * m, m), :],
                local_sem,
            )
            diag.start()
            diag.wait()
        else:
            out_ref[pl.ds(me * m, m), :] = x_ref[:, pl.ds(me * blk, blk)]

        for rdma in rdmas:
            rdma.wait()

    nsem = (N_DEV - 1) * CHUNKS
    space = pltpu.MemorySpace.HBM if MEM == "any" else pltpu.VMEM
    return pl.pallas_call(
        body,
        out_shape=jax.ShapeDtypeStruct((out_rows, blk), x.dtype),
        in_specs=[pl.BlockSpec(memory_space=space)],
        out_specs=pl.BlockSpec(memory_space=space),
        scratch_shapes=[
            pltpu.SemaphoreType.DMA((nsem,)),
            pltpu.SemaphoreType.DMA((nsem,)),
            pltpu.SemaphoreType.REGULAR((2,)),
            pltpu.SemaphoreType.DMA,
        ],
        compiler_params=pltpu.CompilerParams(collective_id=0),
    )(x)


# device time: 12014 ns/iter; 1.8344x vs baseline; 1.8243x over previous
import os

import jax
import jax.numpy as jnp
from jax import lax
from jax.experimental import pallas as pl
from jax.experimental.pallas import tpu as pltpu

N_DEV = 8

ORDER = os.environ.get("A2A_ORDER", "xor")
CHUNKS = int(os.environ.get("A2A_CHUNKS", "1"))
NSENDS = int(os.environ.get("A2A_NSENDS", "7"))
HALF = int(os.environ.get("A2A_HALF", "1"))
BARRIER = os.environ.get("A2A_BARRIER", "auto")
MEM = os.environ.get("A2A_MEM", "vmem")
INTERLEAVE = os.environ.get("A2A_INTERLEAVE", "0") == "1"

_DELTAS = [
    (1, 0, 0), (0, 1, 0), (0, 0, 1),
    (1, 1, 0), (0, 1, 1), (1, 0, 1),
    (1, 1, 1),
]
if os.environ.get("A2A_FAR_FIRST", "0") == "1":
    _DELTAS = _DELTAS[::-1]


def _xor_partner(me, delta):
    dx, dy, dz = delta
    z = me // 4
    r = me - 4 * z
    x = jnp.where((r == 1) | (r == 2), 1, 0)
    y = jnp.where(r >= 2, 1, 0)
    px = x + dx - 2 * x * dx
    py = y + dy - 2 * y * dy
    pz = z + dz - 2 * z * dz
    p4 = 3 * py + px - 2 * px * py
    return 4 * pz + p4


def kernel(x):
    m, n_total = x.shape
    blk = n_total // N_DEV
    out_rows = m * N_DEV
    mc = m // CHUNKS // HALF

    def body(x_ref, out_ref, send_sems, recv_sems, hsems, local_sem):
        me = lax.axis_index("i")
        rdmas = []

        def start_send(j, slot):
            for c in range(CHUNKS):
                s = slot * CHUNKS + c
                rdma = pltpu.make_async_remote_copy(
                    src_ref=x_ref.at[pl.ds(c * mc, mc), pl.ds(j * blk, blk)],
                    dst_ref=out_ref.at[pl.ds(me * m + c * mc, mc), :],
                    send_sem=send_sems.at[s],
                    recv_sem=recv_sems.at[s],
                    device_id=(j,),
                    device_id_type=pl.DeviceIdType.MESH,
                )
                rdma.start()
                rdmas.append(rdma)

        if BARRIER == "peer":
            barrier_sem = pltpu.get_barrier_semaphore()
            psems = [barrier_sem] + [hsems.at[i] for i in range(N_DEV - 2)]
            partners = [_xor_partner(me, d) for d in _DELTAS]
            for k in range(N_DEV - 1):
                pl.semaphore_signal(
                    psems[k], inc=1,
                    device_id=(partners[k],),
                    device_id_type=pl.DeviceIdType.MESH,
                )
            for k in range(N_DEV - 1):
                pl.semaphore_wait(psems[k], 1)
                start_send(partners[k], k)
        elif BARRIER == "explicit":
            barrier_sem = pltpu.get_barrier_semaphore()
            for k in range(1, N_DEV):
                nbr = lax.rem(me + k, N_DEV)
                pl.semaphore_signal(
                    barrier_sem, inc=1,
                    device_id=(nbr,), device_id_type=pl.DeviceIdType.MESH,
                )
            pl.semaphore_wait(barrier_sem, N_DEV - 1)
        elif BARRIER == "hyper":
            barrier_sem = pltpu.get_barrier_semaphore()
            phase_sems = [barrier_sem, hsems.at[0], hsems.at[1]]
            groups = [
                [(1, 0, 0)],
                [(1, 1, 0), (0, 1, 0)],
                [(1, 1, 1), (1, 0, 1), (0, 1, 1), (0, 0, 1)],
            ]
            for p, pdelta in enumerate([(1, 0, 0), (0, 1, 0), (0, 0, 1)]):
                partner = _xor_partner(me, pdelta)
                pl.semaphore_signal(
                    phase_sems[p], inc=1,
                    device_id=(partner,), device_id_type=pl.DeviceIdType.MESH,
                )
                pl.semaphore_wait(phase_sems[p], 1)
                if INTERLEAVE:
                    for d in groups[p]:
                        start_send(_xor_partner(me, d), _DELTAS.index(d))

        if BARRIER != "peer" and not (BARRIER == "hyper" and INTERLEAVE):
            for k in range(1, 1 + NSENDS):
                if ORDER == "xor":
                    j = _xor_partner(me, _DELTAS[k - 1])
                else:
                    j = lax.rem(me + k, N_DEV)
                start_send(j, k - 1)

        if MEM == "any":
            diag = pltpu.make_async_copy(
                x_ref.at[:, pl.ds(me * blk, blk)],
                out_ref.at[pl.ds(me * m, m), :],
                local_sem,
            )
            diag.start()
            diag.wait()
        else:
            out_ref[pl.ds(me * m, m), :] = x_ref[:, pl.ds(me * blk, blk)]

        for rdma in rdmas:
            rdma.wait()

    nsem = (N_DEV - 1) * CHUNKS
    space = pltpu.MemorySpace.HBM if MEM == "any" else pltpu.VMEM
    return pl.pallas_call(
        body,
        out_shape=jax.ShapeDtypeStruct((out_rows, blk), x.dtype),
        in_specs=[pl.BlockSpec(memory_space=space)],
        out_specs=pl.BlockSpec(memory_space=space),
        scratch_shapes=[
            pltpu.SemaphoreType.DMA((nsem,)),
            pltpu.SemaphoreType.DMA((nsem,)),
            pltpu.SemaphoreType.REGULAR((N_DEV - 2,)),
            pltpu.SemaphoreType.DMA,
        ],
        compiler_params=pltpu.CompilerParams(collective_id=0),
    )(x)
